# baseline (speedup 1.0000x reference)
import functools

import jax
import jax.numpy as jnp
from jax import lax
from jax.experimental import pallas as pl
from jax.experimental.pallas import tpu as pltpu
from jax.experimental.pallas import tpu_sc as plsc

B = 16384
NU = 1000000
NI = 100000
NL = 100
NCAT = 1000
D = 64
H = 32

NC = 2
NS = 16
NW = NC * NS
BPW = B // NW

TILE = 512


G = 8


def _sc_gather_body(uidx_h, iidx_h, uemb, iemb, u_out, i_out, idx_v, sem):
  wid = lax.axis_index("s") * NC + lax.axis_index("c")
  base = wid * BPW
  lanes = lax.iota(jnp.int32, 16)
  for idx_h, tab, out in ((uidx_h, uemb, u_out), (iidx_h, iemb, i_out)):
    pltpu.sync_copy(idx_h.at[wid], idx_v)

    def fire(g, carry):
      vec = idx_v[pl.ds(g * 16, 16)]
      for j in range(16):
        r = jnp.sum(jnp.where(lanes == j, vec, 0))
        rb = pl.multiple_of(r, G)
        dst = pl.multiple_of(G * (base + g * 16 + j), G)
        pltpu.async_copy(
            tab.at[pl.ds(rb, G)], out.at[pl.ds(dst, G)], sem)
      return carry

    lax.fori_loop(0, BPW // 16, fire, 0)

    def drain(g, carry):
      pltpu.make_async_copy(
          tab.at[pl.ds(0, G)], out.at[pl.ds(G * base, G)], sem).wait()
      return carry

    lax.fori_loop(0, BPW, drain, 0)


_sc_gather = functools.partial(
    pl.kernel,
    out_type=(
        jax.ShapeDtypeStruct((G * B, D), jnp.bfloat16),
        jax.ShapeDtypeStruct((G * B, D), jnp.bfloat16),
    ),
    mesh=plsc.VectorSubcoreMesh(core_axis_name="c", subcore_axis_name="s"),
    scratch_types=[
        pltpu.VMEM((BPW,), jnp.int32),
        pltpu.SemaphoreType.DMA,
    ],
    compiler_params=pltpu.CompilerParams(needs_layout_passes=False),
)(_sc_gather_body)


def _mlp_body(ub_ref, ib_ref, up_ref, ip_ref, lg_ref, ct_ref,
              lemb_ref, cemb_ref, cwi_ref, cwl_ref, cwc_ref, cb_ref,
              w1u_ref, w1c_ref, b1_ref, w2t_ref, b2_ref, w3t_ref, b3_ref,
              out_ref):
  ub = ub_ref[...].reshape(TILE, G, D)
  ib = ib_ref[...].reshape(TILE, G, D)
  u = jnp.zeros((TILE, D), jnp.float32)
  iv = jnp.zeros((TILE, D), jnp.float32)
  up = up_ref[...]
  ip = ip_ref[...]
  for s in range(G):
    u = u + jnp.where(up == s, 1.0, 0.0) * ub[:, s, :].astype(jnp.float32)
    iv = iv + jnp.where(ip == s, 1.0, 0.0) * ib[:, s, :].astype(jnp.float32)
  lw = lemb_ref[...] @ cwl_ref[...]
  cw2 = cemb_ref[...] @ cwc_ref[...]
  ohl = (lg_ref[...] == lax.broadcasted_iota(jnp.int32, (1, NL), 1)
         ).astype(jnp.float32)
  ohc = (ct_ref[...] == lax.broadcasted_iota(jnp.int32, (1, NCAT), 1)
         ).astype(jnp.float32)
  ic = iv @ cwi_ref[...]
  ic += ohl @ lw
  ic += ohc @ cw2
  ic = jnp.maximum(ic + cb_ref[...], 0.0)
  h1 = u @ w1u_ref[...]
  h1 += ic @ w1c_ref[...]
  h1 = jnp.maximum(h1 + b1_ref[...], 0.0)
  h2 = jnp.maximum(h1 @ w2t_ref[...] + b2_ref[...], 0.0)
  out_ref[...] = h2 @ w3t_ref[...] + b3_ref[...]


def _full(shape):
  return pl.BlockSpec(shape, lambda i: tuple(0 for _ in shape))


_mlp = pl.pallas_call(
    _mlp_body,
    grid=(B // TILE,),
    in_specs=[
        pl.BlockSpec((G * TILE, D), lambda i: (i, 0)),
        pl.BlockSpec((G * TILE, D), lambda i: (i, 0)),
        pl.BlockSpec((TILE, 1), lambda i: (i, 0)),
        pl.BlockSpec((TILE, 1), lambda i: (i, 0)),
        pl.BlockSpec((TILE, 1), lambda i: (i, 0)),
        pl.BlockSpec((TILE, 1), lambda i: (i, 0)),
        _full((NL, H)),
        _full((NCAT, H)),
        _full((D, D)),
        _full((H, D)),
        _full((H, D)),
        _full((1, D)),
        _full((D, 2 * D)),
        _full((D, 2 * D)),
        _full((1, 2 * D)),
        _full((2 * D, D)),
        _full((1, D)),
        _full((D, 1)),
        _full((1, 1)),
    ],
    out_specs=pl.BlockSpec((TILE, 1), lambda i: (i, 0)),
    out_shape=jax.ShapeDtypeStruct((B, 1), jnp.float32),
    compiler_params=pltpu.CompilerParams(
        dimension_semantics=("arbitrary",)),
)


def kernel(user, item, language, category, user_emb, item_emb, language_emb,
           category_emb, cw, cb, w1, b1, w2, b2, w3, b3):
  u_rows, i_rows = _sc_gather(
      ((user // G) * G).reshape(NW, BPW), ((item // G) * G).reshape(NW, BPW),
      user_emb.astype(jnp.bfloat16), item_emb.astype(jnp.bfloat16))
  cwi = cw[:, :D].T
  cwl = cw[:, D:D + H].T
  cwc = cw[:, D + H:].T
  w1u = w1[:, :D].T
  w1c = w1[:, D:].T
  out = _mlp(u_rows, i_rows,
             (user % G).reshape(B, 1), (item % G).reshape(B, 1),
             language.reshape(B, 1), category.reshape(B, 1),
             language_emb, category_emb,
             cwi, cwl, cwc, cb.reshape(1, D),
             w1u, w1c, b1.reshape(1, 2 * D),
             w2.T, b2.reshape(1, D),
             w3.T, b3.reshape(1, 1))
  return out[:, 0]


# speedup vs baseline: 1.0592x; 1.0592x over previous
import functools

import jax
import jax.numpy as jnp
from jax import lax
from jax.experimental import pallas as pl
from jax.experimental.pallas import tpu as pltpu
from jax.experimental.pallas import tpu_sc as plsc

B = 16384
NU = 1000000
NI = 100000
NL = 100
NCAT = 1000
D = 64
H = 32

NC = 2
NS = 16
NW = NC * NS
BPW = B // NW
G = 8

TILE = 512
CK = 8192


def _proj_body(xt_ref, w_ref, out_ref):
  out_ref[...] = lax.dot_general(
      xt_ref[...], w_ref[...], (((0,), (0,)), ((), ())),
      preferred_element_type=jnp.float32).astype(jnp.bfloat16)


def _make_proj(n_rows):
  grid = (n_rows + CK - 1) // CK
  return pl.pallas_call(
      _proj_body,
      grid=(grid,),
      in_specs=[
          pl.BlockSpec((D, CK), lambda i: (0, i)),
          pl.BlockSpec((D, 2 * D), lambda i: (0, 0)),
      ],
      out_specs=pl.BlockSpec((CK, 2 * D), lambda i: (i, 0)),
      out_shape=jax.ShapeDtypeStruct((n_rows, 2 * D), jnp.bfloat16),
      compiler_params=pltpu.CompilerParams(
          dimension_semantics=("arbitrary",)),
  )


_proj_u = _make_proj(NU)
_proj_i = _make_proj(NI)


def _sc_gather_body(uidx_h, iidx_h, utab, itab, u_out, i_out, idx_v, sem):
  wid = lax.axis_index("s") * NC + lax.axis_index("c")
  base = wid * BPW
  lanes = lax.iota(jnp.int32, 16)
  for idx_h, tab, out in ((uidx_h, utab, u_out), (iidx_h, itab, i_out)):
    pltpu.sync_copy(idx_h.at[wid], idx_v)

    def fire(g, carry):
      vec = idx_v[pl.ds(g * 16, 16)]
      for j in range(16):
        r = jnp.sum(jnp.where(lanes == j, vec, 0))
        rb = pl.multiple_of(r, G)
        dst = pl.multiple_of(G * (base + g * 16 + j), G)
        pltpu.async_copy(tab.at[pl.ds(rb, G)], out.at[pl.ds(dst, G)], sem)
      return carry

    lax.fori_loop(0, BPW // 16, fire, 0)

    def drain(g, carry):
      pltpu.make_async_copy(
          tab.at[pl.ds(0, G)], out.at[pl.ds(G * base, G)], sem).wait()
      return carry

    lax.fori_loop(0, BPW, drain, 0)


_sc_gather = functools.partial(
    pl.kernel,
    out_type=(
        jax.ShapeDtypeStruct((G * B, 2 * D), jnp.bfloat16),
        jax.ShapeDtypeStruct((G * B, 2 * D), jnp.bfloat16),
    ),
    mesh=plsc.VectorSubcoreMesh(core_axis_name="c", subcore_axis_name="s"),
    scratch_types=[
        pltpu.VMEM((BPW,), jnp.int32),
        pltpu.SemaphoreType.DMA,
    ],
    compiler_params=pltpu.CompilerParams(needs_layout_passes=False),
)(_sc_gather_body)


def _mlp_body(ub_ref, ib_ref, up_ref, ip_ref, lg_ref, ct_ref,
              lemb_ref, cemb_ref, cwl_ref, cwc_ref, cb_ref,
              w1c_ref, b1_ref, w2t_ref, b2_ref, w3t_ref, b3_ref,
              out_ref):
  ub = ub_ref[...].reshape(TILE, G, 2 * D)
  ib = ib_ref[...].reshape(TILE, G, 2 * D)
  up = up_ref[...]
  ip = ip_ref[...]
  h1u = jnp.zeros((TILE, 2 * D), jnp.float32)
  icp = jnp.zeros((TILE, 2 * D), jnp.float32)
  for s in range(G):
    h1u = h1u + jnp.where(up == s, 1.0, 0.0) * ub[:, s, :].astype(jnp.float32)
    icp = icp + jnp.where(ip == s, 1.0, 0.0) * ib[:, s, :].astype(jnp.float32)
  lw = lemb_ref[...] @ cwl_ref[...]
  cw2 = cemb_ref[...] @ cwc_ref[...]
  ohl = (lg_ref[...] == lax.broadcasted_iota(jnp.int32, (1, NL), 1)
         ).astype(jnp.float32)
  ohc = (ct_ref[...] == lax.broadcasted_iota(jnp.int32, (1, NCAT), 1)
         ).astype(jnp.float32)
  ic = icp[:, :D]
  ic += ohl @ lw
  ic += ohc @ cw2
  ic = jnp.maximum(ic + cb_ref[...], 0.0)
  h1 = h1u
  h1 += ic @ w1c_ref[...]
  h1 = jnp.maximum(h1 + b1_ref[...], 0.0)
  h2 = jnp.maximum(h1 @ w2t_ref[...] + b2_ref[...], 0.0)
  out_ref[...] = h2 @ w3t_ref[...] + b3_ref[...]


def _full(shape):
  return pl.BlockSpec(shape, lambda i: tuple(0 for _ in shape))


_mlp = pl.pallas_call(
    _mlp_body,
    grid=(B // TILE,),
    in_specs=[
        pl.BlockSpec((G * TILE, 2 * D), lambda i: (i, 0)),
        pl.BlockSpec((G * TILE, 2 * D), lambda i: (i, 0)),
        pl.BlockSpec((TILE, 1), lambda i: (i, 0)),
        pl.BlockSpec((TILE, 1), lambda i: (i, 0)),
        pl.BlockSpec((TILE, 1), lambda i: (i, 0)),
        pl.BlockSpec((TILE, 1), lambda i: (i, 0)),
        _full((NL, H)),
        _full((NCAT, H)),
        _full((H, D)),
        _full((H, D)),
        _full((1, D)),
        _full((D, 2 * D)),
        _full((1, 2 * D)),
        _full((2 * D, D)),
        _full((1, D)),
        _full((D, 1)),
        _full((1, 1)),
    ],
    out_specs=pl.BlockSpec((TILE, 1), lambda i: (i, 0)),
    out_shape=jax.ShapeDtypeStruct((B, 1), jnp.float32),
    compiler_params=pltpu.CompilerParams(
        dimension_semantics=("arbitrary",)),
)


def kernel(user, item, language, category, user_emb, item_emb, language_emb,
           category_emb, cw, cb, w1, b1, w2, b2, w3, b3):
  w1u = w1[:, :D].T
  cwi_pad = jnp.pad(cw[:, :D].T, ((0, 0), (0, D)))
  u_proj = _proj_u(user_emb.T, w1u)
  i_proj = _proj_i(item_emb.T, cwi_pad)
  u_rows, i_rows = _sc_gather(
      ((user // G) * G).reshape(NW, BPW), ((item // G) * G).reshape(NW, BPW),
      u_proj, i_proj)
  cwl = cw[:, D:D + H].T
  cwc = cw[:, D + H:].T
  w1c = w1[:, D:].T
  out = _mlp(u_rows, i_rows,
             (user % G).reshape(B, 1), (item % G).reshape(B, 1),
             language.reshape(B, 1), category.reshape(B, 1),
             language_emb, category_emb,
             cwl, cwc, cb.reshape(1, D),
             w1c, b1.reshape(1, 2 * D),
             w2.T, b2.reshape(1, D),
             w3.T, b3.reshape(1, 1))
  return out[:, 0]


# speedup vs baseline: 5.5952x; 5.2826x over previous
import functools

import jax
import jax.numpy as jnp
from jax import lax
from jax.experimental import pallas as pl
from jax.experimental.pallas import tpu as pltpu
from jax.experimental.pallas import tpu_sc as plsc

B = 16384
NU = 1000000
NI = 100000
NL = 100
NCAT = 1000
D = 64
H = 32

NC = 2
NS = 16
NW = NC * NS
BPW = B // NW
G = 8

TILE = 512
CK = 8192


def _proj_body(xt_ref, w_ref, out_ref):
  out_ref[...] = lax.dot_general(
      xt_ref[...], w_ref[...], (((0,), (0,)), ((), ())),
      preferred_element_type=jnp.float32).astype(jnp.bfloat16)


def _make_proj(n_rows):
  grid = (n_rows + CK - 1) // CK
  return pl.pallas_call(
      _proj_body,
      grid=(grid,),
      in_specs=[
          pl.BlockSpec((D, CK), lambda i: (0, i)),
          pl.BlockSpec((D, 2 * D), lambda i: (0, 0)),
      ],
      out_specs=pl.BlockSpec((CK, 2 * D), lambda i: (i, 0)),
      out_shape=jax.ShapeDtypeStruct((n_rows, 2 * D), jnp.bfloat16),
      compiler_params=pltpu.CompilerParams(
          dimension_semantics=("arbitrary",)),
  )


_proj_u = _make_proj(NU)
_proj_i = _make_proj(NI)


NCHUNK = 4
CGRP = BPW // NCHUNK


def _sc_gather_body(uidx_h, iidx_h, utab, itab, u_out, i_out,
                    idx_v, buf, sem):
  wid = lax.axis_index("s") * NC + lax.axis_index("c")
  base = wid * BPW
  lanes = lax.iota(jnp.int32, 16)
  for idx_h, tab, out in ((uidx_h, utab, u_out), (iidx_h, itab, i_out)):
    pltpu.sync_copy(idx_h.at[wid], idx_v)
    for c in range(NCHUNK):
      def fire(g, carry, c=c):
        vec = idx_v[pl.ds(c * CGRP + g * 16, 16)]
        for j in range(16):
          r = jnp.sum(jnp.where(lanes == j, vec, 0))
          rb = pl.multiple_of(r, G)
          dst = pl.multiple_of(G * (g * 16 + j), G)
          pltpu.async_copy(tab.at[pl.ds(rb, G)], buf.at[pl.ds(dst, G)], sem)
        return carry

      lax.fori_loop(0, CGRP // 16, fire, 0)
      pltpu.make_async_copy(tab.at[pl.ds(0, G * CGRP)], buf, sem).wait()
      pltpu.sync_copy(buf, out.at[pl.ds(G * (base + c * CGRP), G * CGRP)])


_sc_gather = functools.partial(
    pl.kernel,
    out_type=(
        jax.ShapeDtypeStruct((G * B, 2 * D), jnp.bfloat16),
        jax.ShapeDtypeStruct((G * B, 2 * D), jnp.bfloat16),
    ),
    mesh=plsc.VectorSubcoreMesh(core_axis_name="c", subcore_axis_name="s"),
    scratch_types=[
        pltpu.VMEM((BPW,), jnp.int32),
        pltpu.VMEM((G * CGRP, 2 * D), jnp.bfloat16),
        pltpu.SemaphoreType.DMA,
    ],
    compiler_params=pltpu.CompilerParams(needs_layout_passes=False),
)(_sc_gather_body)


def _mlp_body(ub_ref, ib_ref, up_ref, ip_ref, lg_ref, ct_ref,
              lemb_ref, cemb_ref, cwl_ref, cwc_ref, cb_ref,
              w1c_ref, b1_ref, w2t_ref, b2_ref, w3t_ref, b3_ref,
              out_ref):
  ub = ub_ref[...].reshape(TILE, G, 2 * D)
  ib = ib_ref[...].reshape(TILE, G, 2 * D)
  up = up_ref[...]
  ip = ip_ref[...]
  h1u = jnp.zeros((TILE, 2 * D), jnp.float32)
  icp = jnp.zeros((TILE, 2 * D), jnp.float32)
  for s in range(G):
    h1u = h1u + jnp.where(up == s, 1.0, 0.0) * ub[:, s, :].astype(jnp.float32)
    icp = icp + jnp.where(ip == s, 1.0, 0.0) * ib[:, s, :].astype(jnp.float32)
  lw = lemb_ref[...] @ cwl_ref[...]
  cw2 = cemb_ref[...] @ cwc_ref[...]
  ohl = (lg_ref[...] == lax.broadcasted_iota(jnp.int32, (1, NL), 1)
         ).astype(jnp.float32)
  ohc = (ct_ref[...] == lax.broadcasted_iota(jnp.int32, (1, NCAT), 1)
         ).astype(jnp.float32)
  ic = icp[:, :D]
  ic += ohl @ lw
  ic += ohc @ cw2
  ic = jnp.maximum(ic + cb_ref[...], 0.0)
  h1 = h1u
  h1 += ic @ w1c_ref[...]
  h1 = jnp.maximum(h1 + b1_ref[...], 0.0)
  h2 = jnp.maximum(h1 @ w2t_ref[...] + b2_ref[...], 0.0)
  out_ref[...] = h2 @ w3t_ref[...] + b3_ref[...]


def _full(shape):
  return pl.BlockSpec(shape, lambda i: tuple(0 for _ in shape))


_mlp = pl.pallas_call(
    _mlp_body,
    grid=(B // TILE,),
    in_specs=[
        pl.BlockSpec((G * TILE, 2 * D), lambda i: (i, 0)),
        pl.BlockSpec((G * TILE, 2 * D), lambda i: (i, 0)),
        pl.BlockSpec((TILE, 1), lambda i: (i, 0)),
        pl.BlockSpec((TILE, 1), lambda i: (i, 0)),
        pl.BlockSpec((TILE, 1), lambda i: (i, 0)),
        pl.BlockSpec((TILE, 1), lambda i: (i, 0)),
        _full((NL, H)),
        _full((NCAT, H)),
        _full((H, D)),
        _full((H, D)),
        _full((1, D)),
        _full((D, 2 * D)),
        _full((1, 2 * D)),
        _full((2 * D, D)),
        _full((1, D)),
        _full((D, 1)),
        _full((1, 1)),
    ],
    out_specs=pl.BlockSpec((TILE, 1), lambda i: (i, 0)),
    out_shape=jax.ShapeDtypeStruct((B, 1), jnp.float32),
    compiler_params=pltpu.CompilerParams(
        dimension_semantics=("arbitrary",)),
)


def kernel(user, item, language, category, user_emb, item_emb, language_emb,
           category_emb, cw, cb, w1, b1, w2, b2, w3, b3):
  w1u = w1[:, :D].T
  cwi_pad = jnp.pad(cw[:, :D].T, ((0, 0), (0, D)))
  u_proj = _proj_u(user_emb.T, w1u)
  i_proj = _proj_i(item_emb.T, cwi_pad)
  u_rows, i_rows = _sc_gather(
      ((user // G) * G).reshape(NW, BPW), ((item // G) * G).reshape(NW, BPW),
      u_proj, i_proj)
  cwl = cw[:, D:D + H].T
  cwc = cw[:, D + H:].T
  w1c = w1[:, D:].T
  out = _mlp(u_rows, i_rows,
             (user % G).reshape(B, 1), (item % G).reshape(B, 1),
             language.reshape(B, 1), category.reshape(B, 1),
             language_emb, category_emb,
             cwl, cwc, cb.reshape(1, D),
             w1c, b1.reshape(1, 2 * D),
             w2.T, b2.reshape(1, D),
             w3.T, b3.reshape(1, 1))
  return out[:, 0]
